# baseline (speedup 1.0000x reference)
import functools
import jax
import jax.numpy as jnp
from jax import lax
from jax.experimental import pallas as pl
from jax.experimental.pallas import tpu as pltpu
from jax.experimental.pallas import tpu_sc as plsc

_K = 64
_ROWS = 64
_COLS = 32768
_NVEC = _COLS // 16
_NC = 2
_NS = 16
_ROWS_PER_TILE = _ROWS // (_NC * _NS)
def _keyify(v):
    b = plsc.bitcast(v, jnp.int32)
    return b ^ (jnp.right_shift(b, 31) & jnp.int32(0x7FFFFFFF))


def _sc_body(x_hbm, o_hbm, row_v, surv_v, hist_v):
    wid = lax.axis_index("s") * _NC + lax.axis_index("c")
    lane = lax.iota(jnp.int32, 16)
    ones16 = jnp.ones((16,), jnp.int32)
    zero16 = jnp.zeros((16,), jnp.int32)
    ninf16 = jnp.full((16,), -jnp.inf, jnp.float32)
    kq = jnp.int32(_K)

    for r in range(_ROWS_PER_TILE):
        row = wid * _ROWS_PER_TILE + r
        pltpu.sync_copy(x_hbm.at[row], row_v)

        for j in range(16):
            hist_v[pl.ds(j * 16, 16)] = zero16

        def hbody(i, carry):
            key = _keyify(row_v[pl.ds(i * 16, 16)])
            digit = jnp.right_shift(key, 24) & jnp.int32(0xFF)
            plsc.addupdate_scatter(hist_v, [digit], ones16)
            return carry

        lax.fori_loop(0, _NVEC, hbody, jnp.int32(0), unroll=8)

        best = jnp.int32(-1)
        run = jnp.int32(0)
        for j in range(15, -1, -1):
            h = hist_v[pl.ds(j * 16, 16)]
            suf = lax.rev(jnp.cumsum(lax.rev(h, (0,))), (0,)) + run
            run = run + jnp.sum(h)
            cand = jnp.where(suf >= kq, lane + jnp.int32(j * 16), jnp.int32(-1))
            best = jnp.maximum(best, jnp.max(cand))
        B = best

        jb = jnp.right_shift(B, 4)
        lb = B & jnp.int32(15)
        hb = hist_v[pl.ds(jb * 16, 16)]
        neg = jnp.int32(-(2**31))
        hist_b = jnp.max(jnp.where(lane == lb, hb, neg))
        count_ge = jnp.int32(0)
        for j in range(16):
            h = hist_v[pl.ds(j * 16, 16)]
            bidx = lane + jnp.int32(j * 16)
            count_ge = count_ge + jnp.sum(jnp.where(bidx >= B, h, jnp.int32(0)))
        k2 = kq - (count_ge - hist_b)

        def cbody(i, off):
            key = _keyify(row_v[pl.ds(i * 16, 16)])
            digit = jnp.right_shift(key, 24) & jnp.int32(0xFF)
            m = digit == B
            plsc.store_compressed(surv_v.at[pl.ds(off, 16)], key, mask=m)
            npop = plsc.all_reduce_population_count(m)
            if getattr(npop, "ndim", 0):
                npop = jnp.max(npop)
            return off + npop

        c = lax.fori_loop(0, _NVEC, cbody, jnp.int32(0), unroll=4)

        base = lax.shift_left(B, jnp.int32(24))
        surv_v[pl.ds(c, 16)] = lax.broadcast_in_dim(base, (16,), ())
        nv = jnp.right_shift(c + jnp.int32(15), 4)

        def bitbody(s, prefix):
            bit = jnp.int32(23) - s
            t = prefix | lax.shift_left(jnp.int32(1), bit)

            def cntb(i, acc):
                kvec = surv_v[pl.ds(i * 16, 16)]
                return acc + jnp.sum((kvec >= t).astype(jnp.int32))

            cnt = lax.fori_loop(0, nv, cntb, jnp.int32(0))
            return jnp.where(cnt >= k2, t, prefix)

        prefix = lax.fori_loop(0, 24, bitbody, base)

        tbits = prefix ^ (jnp.right_shift(prefix, 31) & jnp.int32(0x7FFFFFFF))
        tf = plsc.bitcast(lax.broadcast_in_dim(tbits, (16,), ()), jnp.float32)

        def mbody(i, carry):
            v = row_v[pl.ds(i * 16, 16)]
            row_v[pl.ds(i * 16, 16)] = jnp.where(v < tf, ninf16, v)
            return carry

        lax.fori_loop(0, _NVEC, mbody, jnp.int32(0), unroll=8)
        pltpu.sync_copy(row_v, o_hbm.at[row])


def kernel(x):
    mesh = plsc.VectorSubcoreMesh(core_axis_name="c", subcore_axis_name="s")
    f = functools.partial(
        pl.kernel,
        mesh=mesh,
        out_type=jax.ShapeDtypeStruct((_ROWS, _COLS), jnp.float32),
        compiler_params=pltpu.CompilerParams(needs_layout_passes=False),
        scratch_types=[
            pltpu.VMEM((_COLS,), jnp.float32),
            pltpu.VMEM((_COLS + 16,), jnp.int32),
            pltpu.VMEM((256,), jnp.int32),
        ],
    )(_sc_body)
    return f(x)
